# Initial kernel scaffold; baseline (speedup 1.0000x reference)
import jax
import jax.numpy as jnp
from jax.experimental import pallas as pl


def kernel(ind_2, prop, basis, W_pp1, b_pp1, W_pp2, b_pp2, W_pi, b_pi, W_ii):
    raise NotImplementedError("write your pallas kernel here")


# speedup vs baseline: 2.3875x; 2.3875x over previous
import functools

import jax
import jax.numpy as jnp
from jax import lax
from jax.experimental import pallas as pl
from jax.experimental.pallas import tpu as pltpu
from jax.experimental.pallas import tpu_sc as plsc

N_ATOMS = 50000
N_PAIRS = 800000
D = 64
NB = 10

NC = 2
NS = 16
NW = NC * NS

ROWS_G = 2 * N_PAIRS
G_PER_W = ROWS_G // NW
CHUNK = 400
OPS = 5
OPB = 80
G_CHUNKS = G_PER_W // CHUNK

S_PER_T = N_PAIRS // NS
S_CHUNKS = S_PER_T // CHUNK

HALF = 25088
T_ROWS = HALF // NS
ACC_ROWS = HALF + 8
LAST_ROWS = N_ATOMS - HALF - 15 * T_ROWS


def _mesh():
    return plsc.VectorSubcoreMesh(
        core_axis_name="c", subcore_axis_name="s", num_cores=NC, num_subcores=NS
    )


def _pp_body(x_ref, w1_ref, b1_ref, w2_ref, b2_ref, o_ref):
    x = x_ref[...].astype(jnp.bfloat16)
    h = jnp.tanh(jnp.dot(x, w1_ref[...], preferred_element_type=jnp.float32)
                 + b1_ref[...])
    p = jnp.tanh(jnp.dot(h.astype(jnp.bfloat16), w2_ref[...],
                         preferred_element_type=jnp.float32) + b2_ref[...])
    o_ref[...] = p


def _pp(prop, w1, b1, w2, b2):
    blk = 2000
    grid = N_ATOMS // blk
    return pl.pallas_call(
        _pp_body,
        grid=(grid,),
        in_specs=[
            pl.BlockSpec((blk, D), lambda i: (i, 0)),
            pl.BlockSpec((D, D), lambda i: (0, 0)),
            pl.BlockSpec((1, D), lambda i: (0, 0)),
            pl.BlockSpec((D, D), lambda i: (0, 0)),
            pl.BlockSpec((1, D), lambda i: (0, 0)),
        ],
        out_specs=pl.BlockSpec((blk, D), lambda i: (i, 0)),
        out_shape=jax.ShapeDtypeStruct((N_ATOMS, D), jnp.float32),
    )(prop, w1, b1, w2, b2)


def _gather_body(table_hbm, idx_hbm, out_hbm,
                 i0, i1, i2, i3, i4, rows_v, sem):
    idx_bufs = (i0, i1, i2, i3, i4)
    wid = lax.axis_index("s") * NC + lax.axis_index("c")
    wbase = wid * G_PER_W

    @pl.loop(0, G_CHUNKS)
    def _chunk(ch):
        base = wbase + ch * CHUNK
        for t in range(OPS):
            pltpu.sync_copy(idx_hbm.at[pl.ds(base + t * OPB, OPB)], idx_bufs[t])
        descs = [
            pltpu.async_copy(table_hbm.at[idx_bufs[t]],
                             rows_v.at[pl.ds(t * OPB, OPB)], sem)
            for t in range(OPS)
        ]
        for d in descs:
            d.wait()
        pltpu.sync_copy(rows_v, out_hbm.at[pl.ds(base, CHUNK)])


def _gather(table, idx):
    k = pl.kernel(
        _gather_body,
        out_type=jax.ShapeDtypeStruct((ROWS_G, D), jnp.float32),
        mesh=_mesh(),
        compiler_params=pltpu.CompilerParams(use_tc_tiling_on_sc=False),
        scratch_types=[
            pltpu.VMEM((OPB,), jnp.int32),
            pltpu.VMEM((OPB,), jnp.int32),
            pltpu.VMEM((OPB,), jnp.int32),
            pltpu.VMEM((OPB,), jnp.int32),
            pltpu.VMEM((OPB,), jnp.int32),
            pltpu.VMEM((CHUNK, D), jnp.float32),
            pltpu.SemaphoreType.DMA,
        ],
    )
    return k(table, idx)


def _ffn_body(gi_ref, gj_ref, basis_ref, wpi_ref, bpi_ref, wii_ref, o_ref):
    x = jnp.concatenate([gi_ref[...], gj_ref[...]], axis=1).astype(jnp.bfloat16)
    y = jnp.dot(x, wpi_ref[...], preferred_element_type=jnp.float32)
    y = jnp.tanh(y + bpi_ref[...])
    b = basis_ref[...]
    z = y[:, 0:D] * b[:, 0:1]
    for k in range(1, NB):
        z = z + y[:, k * D:(k + 1) * D] * b[:, k:k + 1]
    o = jnp.tanh(jnp.dot(z.astype(jnp.bfloat16), wii_ref[...],
                         preferred_element_type=jnp.float32))
    o_ref[...] = o


def _ffn(g, basis, wpi, bpi, wii):
    blk = 640
    grid = N_PAIRS // blk
    return pl.pallas_call(
        _ffn_body,
        grid=(grid,),
        in_specs=[
            pl.BlockSpec((blk, D), lambda i: (i, 0)),
            pl.BlockSpec((blk, D), lambda i: (i + N_PAIRS // blk, 0)),
            pl.BlockSpec((blk, NB), lambda i: (i, 0)),
            pl.BlockSpec((2 * D, NB * D), lambda i: (0, 0)),
            pl.BlockSpec((1, NB * D), lambda i: (0, 0)),
            pl.BlockSpec((D, D), lambda i: (0, 0)),
        ],
        out_specs=pl.BlockSpec((blk, D), lambda i: (i, 0)),
        out_shape=jax.ShapeDtypeStruct((N_PAIRS, D), jnp.float32),
    )(g, g, basis, wpi, bpi, wii)


def _scatter_body(ii_hbm, inter_hbm, zeros_hbm, out_hbm,
                  idx_raw, i0, i1, i2, i3, i4, rows_v, acc, sem):
    idx_bufs = (i0, i1, i2, i3, i4)
    c = lax.axis_index("c")
    s = lax.axis_index("s")
    base_atom = c * HALF

    pltpu.sync_copy(zeros_hbm.at[pl.ds(s * T_ROWS, T_ROWS)],
                    acc.at[pl.ds(s * T_ROWS, T_ROWS)])

    @pl.when(s == 0)
    def _():
        pltpu.sync_copy(zeros_hbm.at[pl.ds(HALF, 8)], acc.at[pl.ds(HALF, 8)])

    plsc.subcore_barrier()

    @pl.loop(0, S_CHUNKS)
    def _chunk(ch):
        pbase = s * S_PER_T + ch * CHUNK
        pltpu.sync_copy(ii_hbm.at[pl.ds(pbase, CHUNK)], idx_raw)
        pltpu.sync_copy(inter_hbm.at[pl.ds(pbase, CHUNK)], rows_v)
        for i in range(CHUNK // 16):
            v = idx_raw[pl.ds(i * 16, 16)]
            local = v - base_atom
            ok = (local >= 0) & (local < HALF)
            idx2 = jnp.where(ok, local, HALF)
            idx_bufs[i // OPS][pl.ds((i % OPS) * 16, 16)] = idx2
        for t in range(OPS):
            pltpu.sync_copy(rows_v.at[pl.ds(t * OPB, OPB)],
                            acc.at[idx_bufs[t]], add=True)

    plsc.subcore_barrier()

    is_last = jnp.logical_and(c == NC - 1, s == NS - 1)

    @pl.when(jnp.logical_not(is_last))
    def _():
        pltpu.sync_copy(acc.at[pl.ds(s * T_ROWS, T_ROWS)],
                        out_hbm.at[pl.ds(base_atom + s * T_ROWS, T_ROWS)])

    @pl.when(is_last)
    def _():
        pltpu.sync_copy(acc.at[pl.ds((NS - 1) * T_ROWS, LAST_ROWS)],
                        out_hbm.at[pl.ds(HALF + (NS - 1) * T_ROWS, LAST_ROWS)])


def _scatter(ind_i, inter, zeros):
    k = pl.kernel(
        _scatter_body,
        out_type=jax.ShapeDtypeStruct((N_ATOMS, D), jnp.float32),
        mesh=_mesh(),
        compiler_params=pltpu.CompilerParams(use_tc_tiling_on_sc=False),
        scratch_types=[
            pltpu.VMEM((CHUNK,), jnp.int32),
            pltpu.VMEM((OPB,), jnp.int32),
            pltpu.VMEM((OPB,), jnp.int32),
            pltpu.VMEM((OPB,), jnp.int32),
            pltpu.VMEM((OPB,), jnp.int32),
            pltpu.VMEM((OPB,), jnp.int32),
            pltpu.VMEM((CHUNK, D), jnp.float32),
            pltpu.VMEM_SHARED((ACC_ROWS, D), jnp.float32),
            pltpu.SemaphoreType.DMA,
        ],
    )
    return k(ind_i, inter, zeros)


def kernel(ind_2, prop, basis, W_pp1, b_pp1, W_pp2, b_pp2, W_pi, b_pi, W_ii):
    wpi = (W_pi.reshape(2 * D, D, NB).transpose(0, 2, 1)
           .reshape(2 * D, NB * D).astype(jnp.bfloat16))
    bpi = b_pi.reshape(D, NB).T.reshape(1, NB * D)
    w1 = W_pp1.astype(jnp.bfloat16)
    w2 = W_pp2.astype(jnp.bfloat16)
    b1 = b_pp1.reshape(1, D)
    b2 = b_pp2.reshape(1, D)
    wii = W_ii.astype(jnp.bfloat16)

    ind_flat = ind_2.T.reshape(ROWS_G)
    zeros = jnp.zeros((ACC_ROWS, D), jnp.float32)

    p = _pp(prop, w1, b1, w2, b2)
    g = _gather(p, ind_flat)
    inter = _ffn(g, basis, wpi, bpi, wii)
    out = _scatter(ind_2[:, 0], inter, zeros)
    return out


# speedup vs baseline: 2.8185x; 1.1805x over previous
import functools

import jax
import jax.numpy as jnp
from jax import lax
from jax.experimental import pallas as pl
from jax.experimental.pallas import tpu as pltpu
from jax.experimental.pallas import tpu_sc as plsc

N_ATOMS = 50000
N_PAIRS = 800000
D = 64
NB = 10

NC = 2
NS = 16
NW = NC * NS

ROWS_G = 2 * N_PAIRS
G_PER_W = ROWS_G // NW
CHUNK = 400
OPS = 5
OPB = 80
G_CHUNKS = G_PER_W // CHUNK

S_PER_T = N_PAIRS // NS
S_CHUNKS = S_PER_T // CHUNK

HALF = 25088
T_ROWS = HALF // NS
ACC_ROWS = HALF + 8
LAST_ROWS = N_ATOMS - HALF - 15 * T_ROWS


def _mesh():
    return plsc.VectorSubcoreMesh(
        core_axis_name="c", subcore_axis_name="s", num_cores=NC, num_subcores=NS
    )


def _pp_body(x_ref, w1_ref, b1_ref, w2_ref, b2_ref, o_ref):
    x = x_ref[...].astype(jnp.bfloat16)
    h = jnp.tanh(jnp.dot(x, w1_ref[...], preferred_element_type=jnp.float32)
                 + b1_ref[...])
    p = jnp.tanh(jnp.dot(h.astype(jnp.bfloat16), w2_ref[...],
                         preferred_element_type=jnp.float32) + b2_ref[...])
    o_ref[...] = p


def _pp(prop, w1, b1, w2, b2):
    blk = 2000
    grid = N_ATOMS // blk
    return pl.pallas_call(
        _pp_body,
        grid=(grid,),
        in_specs=[
            pl.BlockSpec((blk, D), lambda i: (i, 0)),
            pl.BlockSpec((D, D), lambda i: (0, 0)),
            pl.BlockSpec((1, D), lambda i: (0, 0)),
            pl.BlockSpec((D, D), lambda i: (0, 0)),
            pl.BlockSpec((1, D), lambda i: (0, 0)),
        ],
        out_specs=pl.BlockSpec((blk, D), lambda i: (i, 0)),
        out_shape=jax.ShapeDtypeStruct((N_ATOMS, D), jnp.float32),
    )(prop, w1, b1, w2, b2)


def _gather_body(table_hbm, idx_hbm, out_hbm,
                 i0, i1, i2, i3, i4, rows_v, sem):
    idx_bufs = (i0, i1, i2, i3, i4)
    wid = lax.axis_index("s") * NC + lax.axis_index("c")
    wbase = wid * G_PER_W

    @pl.loop(0, G_CHUNKS)
    def _chunk(ch):
        base = wbase + ch * CHUNK
        for t in range(OPS):
            pltpu.sync_copy(idx_hbm.at[pl.ds(base + t * OPB, OPB)], idx_bufs[t])
        descs = [
            pltpu.async_copy(table_hbm.at[idx_bufs[t]],
                             rows_v.at[pl.ds(t * OPB, OPB)], sem)
            for t in range(OPS)
        ]
        for d in descs:
            d.wait()
        pltpu.sync_copy(rows_v, out_hbm.at[pl.ds(base, CHUNK)])


def _gather(table, idx):
    k = pl.kernel(
        _gather_body,
        out_type=jax.ShapeDtypeStruct((ROWS_G, D), jnp.float32),
        mesh=_mesh(),
        compiler_params=pltpu.CompilerParams(use_tc_tiling_on_sc=False),
        scratch_types=[
            pltpu.VMEM((OPB,), jnp.int32),
            pltpu.VMEM((OPB,), jnp.int32),
            pltpu.VMEM((OPB,), jnp.int32),
            pltpu.VMEM((OPB,), jnp.int32),
            pltpu.VMEM((OPB,), jnp.int32),
            pltpu.VMEM((CHUNK, D), jnp.float32),
            pltpu.SemaphoreType.DMA,
        ],
    )
    return k(table, idx)


def _ffn_body(gi_ref, gj_ref, basis_ref, wpi_ref, bpi_ref, e_ref, wii10_ref,
              o_ref):
    x = jnp.concatenate([gi_ref[...], gj_ref[...]], axis=1).astype(jnp.bfloat16)
    y = jnp.tanh(jnp.dot(x, wpi_ref[...], preferred_element_type=jnp.float32)
                 + bpi_ref[...])
    bexp = jnp.dot(basis_ref[...].astype(jnp.bfloat16), e_ref[...],
                   preferred_element_type=jnp.float32)
    yp = (y * bexp).astype(jnp.bfloat16)
    o = jnp.tanh(jnp.dot(yp, wii10_ref[...], preferred_element_type=jnp.float32))
    o_ref[...] = o


def _ffn(g, basis, wpi, bpi, e, wii10):
    blk = 640
    grid = N_PAIRS // blk
    return pl.pallas_call(
        _ffn_body,
        grid=(grid,),
        in_specs=[
            pl.BlockSpec((blk, D), lambda i: (i, 0)),
            pl.BlockSpec((blk, D), lambda i: (i + N_PAIRS // blk, 0)),
            pl.BlockSpec((blk, NB), lambda i: (i, 0)),
            pl.BlockSpec((2 * D, NB * D), lambda i: (0, 0)),
            pl.BlockSpec((1, NB * D), lambda i: (0, 0)),
            pl.BlockSpec((NB, NB * D), lambda i: (0, 0)),
            pl.BlockSpec((NB * D, D), lambda i: (0, 0)),
        ],
        out_specs=pl.BlockSpec((blk, D), lambda i: (i, 0)),
        out_shape=jax.ShapeDtypeStruct((N_PAIRS, D), jnp.float32),
    )(g, g, basis, wpi, bpi, e, wii10)


def _scatter_body(ii_hbm, inter_hbm, zeros_hbm, out_hbm,
                  idx_raw, i0, i1, i2, i3, i4, rows_v, acc, sem):
    idx_bufs = (i0, i1, i2, i3, i4)
    c = lax.axis_index("c")
    s = lax.axis_index("s")
    base_atom = c * HALF

    pltpu.sync_copy(zeros_hbm.at[pl.ds(s * T_ROWS, T_ROWS)],
                    acc.at[pl.ds(s * T_ROWS, T_ROWS)])

    @pl.when(s == 0)
    def _():
        pltpu.sync_copy(zeros_hbm.at[pl.ds(HALF, 8)], acc.at[pl.ds(HALF, 8)])

    plsc.subcore_barrier()

    @pl.loop(0, S_CHUNKS)
    def _chunk(ch):
        pbase = s * S_PER_T + ch * CHUNK
        pltpu.sync_copy(ii_hbm.at[pl.ds(pbase, CHUNK)], idx_raw)
        pltpu.sync_copy(inter_hbm.at[pl.ds(pbase, CHUNK)], rows_v)
        for i in range(CHUNK // 16):
            v = idx_raw[pl.ds(i * 16, 16)]
            local = v - base_atom
            ok = (local >= 0) & (local < HALF)
            idx2 = jnp.where(ok, local, HALF)
            idx_bufs[i // OPS][pl.ds((i % OPS) * 16, 16)] = idx2
        for t in range(OPS):
            pltpu.sync_copy(rows_v.at[pl.ds(t * OPB, OPB)],
                            acc.at[idx_bufs[t]], add=True)

    plsc.subcore_barrier()

    is_last = jnp.logical_and(c == NC - 1, s == NS - 1)

    @pl.when(jnp.logical_not(is_last))
    def _():
        pltpu.sync_copy(acc.at[pl.ds(s * T_ROWS, T_ROWS)],
                        out_hbm.at[pl.ds(base_atom + s * T_ROWS, T_ROWS)])

    @pl.when(is_last)
    def _():
        pltpu.sync_copy(acc.at[pl.ds((NS - 1) * T_ROWS, LAST_ROWS)],
                        out_hbm.at[pl.ds(HALF + (NS - 1) * T_ROWS, LAST_ROWS)])


def _scatter(ind_i, inter, zeros):
    k = pl.kernel(
        _scatter_body,
        out_type=jax.ShapeDtypeStruct((N_ATOMS, D), jnp.float32),
        mesh=_mesh(),
        compiler_params=pltpu.CompilerParams(use_tc_tiling_on_sc=False),
        scratch_types=[
            pltpu.VMEM((CHUNK,), jnp.int32),
            pltpu.VMEM((OPB,), jnp.int32),
            pltpu.VMEM((OPB,), jnp.int32),
            pltpu.VMEM((OPB,), jnp.int32),
            pltpu.VMEM((OPB,), jnp.int32),
            pltpu.VMEM((OPB,), jnp.int32),
            pltpu.VMEM((CHUNK, D), jnp.float32),
            pltpu.VMEM_SHARED((ACC_ROWS, D), jnp.float32),
            pltpu.SemaphoreType.DMA,
        ],
    )
    return k(ind_i, inter, zeros)


def kernel(ind_2, prop, basis, W_pp1, b_pp1, W_pp2, b_pp2, W_pi, b_pi, W_ii):
    wpi = (W_pi.reshape(2 * D, D, NB).transpose(0, 2, 1)
           .reshape(2 * D, NB * D).astype(jnp.bfloat16))
    bpi = b_pi.reshape(D, NB).T.reshape(1, NB * D)
    w1 = W_pp1.astype(jnp.bfloat16)
    w2 = W_pp2.astype(jnp.bfloat16)
    b1 = b_pp1.reshape(1, D)
    b2 = b_pp2.reshape(1, D)
    e = jnp.repeat(jnp.eye(NB, dtype=jnp.bfloat16), D, axis=1)
    wii10 = jnp.tile(W_ii, (NB, 1)).astype(jnp.bfloat16)

    ind_flat = ind_2.T.reshape(ROWS_G)
    zeros = jnp.zeros((ACC_ROWS, D), jnp.float32)

    p = _pp(prop, w1, b1, w2, b2)
    g = _gather(p, ind_flat)
    inter = _ffn(g, basis, wpi, bpi, e, wii10)
    out = _scatter(ind_2[:, 0], inter, zeros)
    return out


# speedup vs baseline: 3.6019x; 1.2779x over previous
import functools

import jax
import jax.numpy as jnp
from jax import lax
from jax.experimental import pallas as pl
from jax.experimental.pallas import tpu as pltpu
from jax.experimental.pallas import tpu_sc as plsc

N_ATOMS = 50000
N_PAIRS = 800000
D = 64
NB = 10

NC = 2
NS = 16
NW = NC * NS

ROWS_G = 2 * N_PAIRS
G_PER_W = ROWS_G // NW
CHUNK = 400
OPS = 5
OPB = 80
G_CHUNKS = G_PER_W // CHUNK

S_PER_T = N_PAIRS // NS
S_CHUNKS = S_PER_T // CHUNK

HALF = 25088
T_ROWS = HALF // NS
ACC_ROWS = HALF + 8
LAST_ROWS = N_ATOMS - HALF - 15 * T_ROWS


def _mesh():
    return plsc.VectorSubcoreMesh(
        core_axis_name="c", subcore_axis_name="s", num_cores=NC, num_subcores=NS
    )


def _pp_body(x_ref, w1_ref, b1_ref, w2_ref, b2_ref, o_ref):
    x = x_ref[...].astype(jnp.bfloat16)
    h = jnp.tanh(jnp.dot(x, w1_ref[...], preferred_element_type=jnp.float32)
                 + b1_ref[...])
    p = jnp.tanh(jnp.dot(h.astype(jnp.bfloat16), w2_ref[...],
                         preferred_element_type=jnp.float32) + b2_ref[...])
    o_ref[...] = p


def _pp(prop, w1, b1, w2, b2):
    blk = 2000
    grid = N_ATOMS // blk
    return pl.pallas_call(
        _pp_body,
        grid=(grid,),
        in_specs=[
            pl.BlockSpec((blk, D), lambda i: (i, 0)),
            pl.BlockSpec((D, D), lambda i: (0, 0)),
            pl.BlockSpec((1, D), lambda i: (0, 0)),
            pl.BlockSpec((D, D), lambda i: (0, 0)),
            pl.BlockSpec((1, D), lambda i: (0, 0)),
        ],
        out_specs=pl.BlockSpec((blk, D), lambda i: (i, 0)),
        out_shape=jax.ShapeDtypeStruct((N_ATOMS, D), jnp.float32),
    )(prop, w1, b1, w2, b2)


def _gather_body(table_hbm, idx_hbm, out_hbm,
                 i0, i1, i2, i3, i4, rows_v, sem):
    idx_bufs = (i0, i1, i2, i3, i4)
    wid = lax.axis_index("s") * NC + lax.axis_index("c")
    wbase = wid * G_PER_W
    is_j = wbase >= N_PAIRS
    col = jnp.where(is_j, D, 0)
    pbase = wbase - jnp.where(is_j, N_PAIRS, 0)

    @pl.loop(0, G_CHUNKS)
    def _chunk(ch):
        base = wbase + ch * CHUNK
        for t in range(OPS):
            pltpu.sync_copy(idx_hbm.at[pl.ds(base + t * OPB, OPB)], idx_bufs[t])
        descs = [
            pltpu.async_copy(table_hbm.at[idx_bufs[t]],
                             rows_v.at[pl.ds(t * OPB, OPB)], sem)
            for t in range(OPS)
        ]
        for d in descs:
            d.wait()
        pltpu.sync_copy(rows_v,
                        out_hbm.at[pl.ds(pbase + ch * CHUNK, CHUNK),
                                   pl.ds(col, D)])


def _gather(table, idx):
    k = pl.kernel(
        _gather_body,
        out_type=jax.ShapeDtypeStruct((N_PAIRS, 2 * D), jnp.float32),
        mesh=_mesh(),
        compiler_params=pltpu.CompilerParams(use_tc_tiling_on_sc=False),
        scratch_types=[
            pltpu.VMEM((OPB,), jnp.int32),
            pltpu.VMEM((OPB,), jnp.int32),
            pltpu.VMEM((OPB,), jnp.int32),
            pltpu.VMEM((OPB,), jnp.int32),
            pltpu.VMEM((OPB,), jnp.int32),
            pltpu.VMEM((CHUNK, D), jnp.float32),
            pltpu.SemaphoreType.DMA,
        ],
    )
    return k(table, idx)


def _ffn_body(g_ref, bt_ref, wpi_ref, bpi_ref, e_ref, wii10_ref, o_ref):
    x = g_ref[...].astype(jnp.bfloat16)
    y = jnp.tanh(jnp.dot(x, wpi_ref[...], preferred_element_type=jnp.float32)
                 + bpi_ref[...])
    bexp = lax.dot_general(bt_ref[...].astype(jnp.bfloat16), e_ref[...],
                           (((0,), (0,)), ((), ())),
                           preferred_element_type=jnp.float32)
    yp = (y * bexp).astype(jnp.bfloat16)
    o = jnp.tanh(jnp.dot(yp, wii10_ref[...], preferred_element_type=jnp.float32))
    o_ref[...] = o


def _ffn(g, basis_t, wpi, bpi, e, wii10):
    blk = 640
    grid = N_PAIRS // blk
    return pl.pallas_call(
        _ffn_body,
        grid=(grid,),
        in_specs=[
            pl.BlockSpec((blk, 2 * D), lambda i: (i, 0)),
            pl.BlockSpec((NB, blk), lambda i: (0, i)),
            pl.BlockSpec((2 * D, NB * D), lambda i: (0, 0)),
            pl.BlockSpec((1, NB * D), lambda i: (0, 0)),
            pl.BlockSpec((NB, NB * D), lambda i: (0, 0)),
            pl.BlockSpec((NB * D, D), lambda i: (0, 0)),
        ],
        out_specs=pl.BlockSpec((blk, D), lambda i: (i, 0)),
        out_shape=jax.ShapeDtypeStruct((N_PAIRS, D), jnp.float32),
    )(g, basis_t, wpi, bpi, e, wii10)


def _scatter_body(ii_hbm, inter_hbm, zeros_hbm, out_hbm,
                  idx_raw, i0, i1, i2, i3, i4, rows_v, acc, sem):
    idx_bufs = (i0, i1, i2, i3, i4)
    c = lax.axis_index("c")
    s = lax.axis_index("s")
    base_atom = c * HALF

    pltpu.sync_copy(zeros_hbm.at[pl.ds(s * T_ROWS, T_ROWS)],
                    acc.at[pl.ds(s * T_ROWS, T_ROWS)])

    @pl.when(s == 0)
    def _():
        pltpu.sync_copy(zeros_hbm.at[pl.ds(HALF, 8)], acc.at[pl.ds(HALF, 8)])

    plsc.subcore_barrier()

    @pl.loop(0, S_CHUNKS)
    def _chunk(ch):
        pbase = s * S_PER_T + ch * CHUNK
        pltpu.sync_copy(ii_hbm.at[pl.ds(pbase, CHUNK)], idx_raw)
        pltpu.sync_copy(inter_hbm.at[pl.ds(pbase, CHUNK)], rows_v)
        for i in range(CHUNK // 16):
            v = idx_raw[pl.ds(i * 16, 16)]
            local = v - base_atom
            ok = (local >= 0) & (local < HALF)
            idx2 = jnp.where(ok, local, HALF)
            idx_bufs[i // OPS][pl.ds((i % OPS) * 16, 16)] = idx2
        for t in range(OPS):
            pltpu.sync_copy(rows_v.at[pl.ds(t * OPB, OPB)],
                            acc.at[idx_bufs[t]], add=True)

    plsc.subcore_barrier()

    is_last = jnp.logical_and(c == NC - 1, s == NS - 1)

    @pl.when(jnp.logical_not(is_last))
    def _():
        pltpu.sync_copy(acc.at[pl.ds(s * T_ROWS, T_ROWS)],
                        out_hbm.at[pl.ds(base_atom + s * T_ROWS, T_ROWS)])

    @pl.when(is_last)
    def _():
        pltpu.sync_copy(acc.at[pl.ds((NS - 1) * T_ROWS, LAST_ROWS)],
                        out_hbm.at[pl.ds(HALF + (NS - 1) * T_ROWS, LAST_ROWS)])


def _scatter(ind_i, inter, zeros):
    k = pl.kernel(
        _scatter_body,
        out_type=jax.ShapeDtypeStruct((N_ATOMS, D), jnp.float32),
        mesh=_mesh(),
        compiler_params=pltpu.CompilerParams(use_tc_tiling_on_sc=False),
        scratch_types=[
            pltpu.VMEM((CHUNK,), jnp.int32),
            pltpu.VMEM((OPB,), jnp.int32),
            pltpu.VMEM((OPB,), jnp.int32),
            pltpu.VMEM((OPB,), jnp.int32),
            pltpu.VMEM((OPB,), jnp.int32),
            pltpu.VMEM((OPB,), jnp.int32),
            pltpu.VMEM((CHUNK, D), jnp.float32),
            pltpu.VMEM_SHARED((ACC_ROWS, D), jnp.float32),
            pltpu.SemaphoreType.DMA,
        ],
    )
    return k(ind_i, inter, zeros)


def kernel(ind_2, prop, basis, W_pp1, b_pp1, W_pp2, b_pp2, W_pi, b_pi, W_ii):
    wpi = (W_pi.reshape(2 * D, D, NB).transpose(0, 2, 1)
           .reshape(2 * D, NB * D).astype(jnp.bfloat16))
    bpi = b_pi.reshape(D, NB).T.reshape(1, NB * D)
    w1 = W_pp1.astype(jnp.bfloat16)
    w2 = W_pp2.astype(jnp.bfloat16)
    b1 = b_pp1.reshape(1, D)
    b2 = b_pp2.reshape(1, D)
    e = jnp.repeat(jnp.eye(NB, dtype=jnp.bfloat16), D, axis=1)
    wii10 = jnp.tile(W_ii, (NB, 1)).astype(jnp.bfloat16)

    ind_flat = ind_2.T.reshape(ROWS_G)
    zeros = jnp.zeros((ACC_ROWS, D), jnp.float32)

    p = _pp(prop, w1, b1, w2, b2)
    g = _gather(p, ind_flat)
    inter = _ffn(g, basis.T, wpi, bpi, e, wii10)
    out = _scatter(ind_2[:, 0], inter, zeros)
    return out


# speedup vs baseline: 4.5941x; 1.2755x over previous
import functools

import jax
import jax.numpy as jnp
from jax import lax
from jax.experimental import pallas as pl
from jax.experimental.pallas import tpu as pltpu
from jax.experimental.pallas import tpu_sc as plsc

N_ATOMS = 50000
N_PAIRS = 800000
D = 64
NB = 10

NC = 2
NS = 16
NW = NC * NS

ROWS_G = 2 * N_PAIRS
G_PER_W = ROWS_G // NW
CHUNK = 400
OPS = 5
OPB = 80
G_CHUNKS = G_PER_W // CHUNK

S_PER_T = N_PAIRS // NS
S_CHUNKS = S_PER_T // CHUNK

HALF = 25088
T_ROWS = HALF // NS
ACC_ROWS = HALF + 8
LAST_ROWS = N_ATOMS - HALF - 15 * T_ROWS


def _mesh():
    return plsc.VectorSubcoreMesh(
        core_axis_name="c", subcore_axis_name="s", num_cores=NC, num_subcores=NS
    )


def _pp_body(x_ref, w1_ref, b1_ref, w2_ref, b2_ref, o_ref):
    x = x_ref[...].astype(jnp.bfloat16)
    h = jnp.tanh(jnp.dot(x, w1_ref[...], preferred_element_type=jnp.float32)
                 + b1_ref[...])
    p = jnp.tanh(jnp.dot(h.astype(jnp.bfloat16), w2_ref[...],
                         preferred_element_type=jnp.float32) + b2_ref[...])
    o_ref[...] = p


def _pp(prop, w1, b1, w2, b2):
    blk = 2000
    grid = N_ATOMS // blk
    return pl.pallas_call(
        _pp_body,
        grid=(grid,),
        in_specs=[
            pl.BlockSpec((blk, D), lambda i: (i, 0)),
            pl.BlockSpec((D, D), lambda i: (0, 0)),
            pl.BlockSpec((1, D), lambda i: (0, 0)),
            pl.BlockSpec((D, D), lambda i: (0, 0)),
            pl.BlockSpec((1, D), lambda i: (0, 0)),
        ],
        out_specs=pl.BlockSpec((blk, D), lambda i: (i, 0)),
        out_shape=jax.ShapeDtypeStruct((N_ATOMS, D), jnp.float32),
    )(prop, w1, b1, w2, b2)


def _gather_body(table_hbm, idx_hbm, out_hbm,
                 i0, i1, i2, i3, i4, rows_v, sem):
    idx_bufs = (i0, i1, i2, i3, i4)
    wid = lax.axis_index("s") * NC + lax.axis_index("c")
    wbase = wid * G_PER_W
    is_j = wbase >= N_PAIRS
    col = jnp.where(is_j, D, 0)
    pbase = wbase - jnp.where(is_j, N_PAIRS, 0)

    @pl.loop(0, G_CHUNKS)
    def _chunk(ch):
        base = wbase + ch * CHUNK
        for t in range(OPS):
            pltpu.sync_copy(idx_hbm.at[pl.ds(base + t * OPB, OPB)], idx_bufs[t])
        descs = [
            pltpu.async_copy(table_hbm.at[idx_bufs[t]],
                             rows_v.at[pl.ds(t * OPB, OPB)], sem)
            for t in range(OPS)
        ]
        for d in descs:
            d.wait()
        pltpu.sync_copy(rows_v,
                        out_hbm.at[pl.ds(pbase + ch * CHUNK, CHUNK),
                                   pl.ds(col, D)])


def _gather(table, idx):
    k = pl.kernel(
        _gather_body,
        out_type=jax.ShapeDtypeStruct((N_PAIRS, 2 * D), jnp.float32),
        mesh=_mesh(),
        compiler_params=pltpu.CompilerParams(use_tc_tiling_on_sc=False),
        scratch_types=[
            pltpu.VMEM((OPB,), jnp.int32),
            pltpu.VMEM((OPB,), jnp.int32),
            pltpu.VMEM((OPB,), jnp.int32),
            pltpu.VMEM((OPB,), jnp.int32),
            pltpu.VMEM((OPB,), jnp.int32),
            pltpu.VMEM((CHUNK, D), jnp.float32),
            pltpu.SemaphoreType.DMA,
        ],
    )
    return k(table, idx)


def _ffn_body(g_ref, bt_ref, wpi_ref, bpi_ref, e_ref, wii10_ref, o_ref):
    x = g_ref[...].astype(jnp.bfloat16)
    y = jnp.tanh(jnp.dot(x, wpi_ref[...], preferred_element_type=jnp.float32)
                 + bpi_ref[...])
    bexp = lax.dot_general(bt_ref[...].astype(jnp.bfloat16), e_ref[...],
                           (((0,), (0,)), ((), ())),
                           preferred_element_type=jnp.float32)
    yp = (y * bexp).astype(jnp.bfloat16)
    o = jnp.tanh(jnp.dot(yp, wii10_ref[...], preferred_element_type=jnp.float32))
    o_ref[:, 0:D] = o


def _ffn(g, basis_t, wpi, bpi, e, wii10):
    blk = 1280
    grid = N_PAIRS // blk
    return pl.pallas_call(
        _ffn_body,
        grid=(grid,),
        in_specs=[
            pl.BlockSpec((blk, 2 * D), lambda i: (i, 0)),
            pl.BlockSpec((NB, blk), lambda i: (0, i)),
            pl.BlockSpec((2 * D, NB * D), lambda i: (0, 0)),
            pl.BlockSpec((1, NB * D), lambda i: (0, 0)),
            pl.BlockSpec((NB, NB * D), lambda i: (0, 0)),
            pl.BlockSpec((NB * D, D), lambda i: (0, 0)),
        ],
        out_specs=pl.BlockSpec((blk, 2 * D), lambda i: (i, 0)),
        out_shape=jax.ShapeDtypeStruct((N_PAIRS, 2 * D), jnp.float32),
    )(g, basis_t, wpi, bpi, e, wii10)


def _scatter_body(ii_hbm, inter_hbm, zeros_hbm, out_hbm,
                  idx_raw, i0, i1, i2, i3, i4, rows_v, acc, sem):
    idx_bufs = (i0, i1, i2, i3, i4)
    c = lax.axis_index("c")
    s = lax.axis_index("s")
    base_atom = c * HALF

    pltpu.sync_copy(zeros_hbm.at[pl.ds(s * T_ROWS, T_ROWS)],
                    acc.at[pl.ds(s * T_ROWS, T_ROWS)])

    @pl.when(s == 0)
    def _():
        pltpu.sync_copy(zeros_hbm.at[pl.ds(HALF, 8)], acc.at[pl.ds(HALF, 8)])

    plsc.subcore_barrier()

    @pl.loop(0, S_CHUNKS)
    def _chunk(ch):
        pbase = s * S_PER_T + ch * CHUNK
        pltpu.sync_copy(ii_hbm.at[pl.ds(pbase, CHUNK)], idx_raw)
        pltpu.sync_copy(inter_hbm.at[pl.ds(pbase, CHUNK), pl.ds(0, D)], rows_v)
        for i in range(CHUNK // 16):
            v = idx_raw[pl.ds(i * 16, 16)]
            local = v - base_atom
            ok = (local >= 0) & (local < HALF)
            idx2 = jnp.where(ok, local, HALF)
            idx_bufs[i // OPS][pl.ds((i % OPS) * 16, 16)] = idx2
        for t in range(OPS):
            pltpu.sync_copy(rows_v.at[pl.ds(t * OPB, OPB)],
                            acc.at[idx_bufs[t]], add=True)

    plsc.subcore_barrier()

    is_last = jnp.logical_and(c == NC - 1, s == NS - 1)

    @pl.when(jnp.logical_not(is_last))
    def _():
        pltpu.sync_copy(acc.at[pl.ds(s * T_ROWS, T_ROWS)],
                        out_hbm.at[pl.ds(base_atom + s * T_ROWS, T_ROWS)])

    @pl.when(is_last)
    def _():
        pltpu.sync_copy(acc.at[pl.ds((NS - 1) * T_ROWS, LAST_ROWS)],
                        out_hbm.at[pl.ds(HALF + (NS - 1) * T_ROWS, LAST_ROWS)])


def _scatter(ind_i, inter, zeros):
    k = pl.kernel(
        _scatter_body,
        out_type=jax.ShapeDtypeStruct((N_ATOMS, D), jnp.float32),
        mesh=_mesh(),
        compiler_params=pltpu.CompilerParams(use_tc_tiling_on_sc=False),
        scratch_types=[
            pltpu.VMEM((CHUNK,), jnp.int32),
            pltpu.VMEM((OPB,), jnp.int32),
            pltpu.VMEM((OPB,), jnp.int32),
            pltpu.VMEM((OPB,), jnp.int32),
            pltpu.VMEM((OPB,), jnp.int32),
            pltpu.VMEM((OPB,), jnp.int32),
            pltpu.VMEM((CHUNK, D), jnp.float32),
            pltpu.VMEM_SHARED((ACC_ROWS, D), jnp.float32),
            pltpu.SemaphoreType.DMA,
        ],
    )
    return k(ind_i, inter, zeros)


def kernel(ind_2, prop, basis, W_pp1, b_pp1, W_pp2, b_pp2, W_pi, b_pi, W_ii):
    wpi = (W_pi.reshape(2 * D, D, NB).transpose(0, 2, 1)
           .reshape(2 * D, NB * D).astype(jnp.bfloat16))
    bpi = b_pi.reshape(D, NB).T.reshape(1, NB * D)
    w1 = W_pp1.astype(jnp.bfloat16)
    w2 = W_pp2.astype(jnp.bfloat16)
    b1 = b_pp1.reshape(1, D)
    b2 = b_pp2.reshape(1, D)
    e = jnp.repeat(jnp.eye(NB, dtype=jnp.bfloat16), D, axis=1)
    wii10 = jnp.tile(W_ii, (NB, 1)).astype(jnp.bfloat16)

    ind_flat = ind_2.T.reshape(ROWS_G)
    zeros = jnp.zeros((ACC_ROWS, D), jnp.float32)

    p = _pp(prop, w1, b1, w2, b2)
    g = _gather(p, ind_flat)
    inter = _ffn(g, basis.T, wpi, bpi, e, wii10)
    out = _scatter(ind_2[:, 0], inter, zeros)
    return out


# speedup vs baseline: 5.3946x; 1.1743x over previous
import functools

import jax
import jax.numpy as jnp
from jax import lax
from jax.experimental import pallas as pl
from jax.experimental.pallas import tpu as pltpu
from jax.experimental.pallas import tpu_sc as plsc

N_ATOMS = 50000
N_PAIRS = 800000
D = 64
NB = 10

NC = 2
NS = 16
NW = NC * NS

ROWS_G = 2 * N_PAIRS
G_PER_W = ROWS_G // NW
CHUNK = 400
OPS = 5
OPB = 80
G_CHUNKS = G_PER_W // CHUNK

S_PER_T = N_PAIRS // NS
SCH = 80
S_CHUNKS = S_PER_T // SCH

HALF = 25088
T_ROWS = HALF // NS
ACC_ROWS = HALF + 8
LAST_ROWS = N_ATOMS - HALF - 15 * T_ROWS


def _mesh():
    return plsc.VectorSubcoreMesh(
        core_axis_name="c", subcore_axis_name="s", num_cores=NC, num_subcores=NS
    )


def _pp_body(x_ref, w1_ref, b1_ref, w2_ref, b2_ref, o_ref):
    x = x_ref[...].astype(jnp.bfloat16)
    h = jnp.tanh(jnp.dot(x, w1_ref[...], preferred_element_type=jnp.float32)
                 + b1_ref[...])
    p = jnp.tanh(jnp.dot(h.astype(jnp.bfloat16), w2_ref[...],
                         preferred_element_type=jnp.float32) + b2_ref[...])
    o_ref[...] = p


def _pp(prop, w1, b1, w2, b2):
    blk = 2000
    grid = N_ATOMS // blk
    return pl.pallas_call(
        _pp_body,
        grid=(grid,),
        in_specs=[
            pl.BlockSpec((blk, D), lambda i: (i, 0)),
            pl.BlockSpec((D, D), lambda i: (0, 0)),
            pl.BlockSpec((1, D), lambda i: (0, 0)),
            pl.BlockSpec((D, D), lambda i: (0, 0)),
            pl.BlockSpec((1, D), lambda i: (0, 0)),
        ],
        out_specs=pl.BlockSpec((blk, D), lambda i: (i, 0)),
        out_shape=jax.ShapeDtypeStruct((N_ATOMS, D), jnp.float32),
    )(prop, w1, b1, w2, b2)


def _gather_body(table_hbm, idx_hbm, out_hbm,
                 idx_v, rows_a, rows_b, gsem_a, gsem_b, wsem_a, wsem_b):
    rows = (rows_a, rows_b)
    gsem = (gsem_a, gsem_b)
    wsem = (wsem_a, wsem_b)
    wid = lax.axis_index("s") * NC + lax.axis_index("c")
    wbase = wid * G_PER_W
    is_j = wbase >= N_PAIRS
    col = jnp.where(is_j, D, 0)
    pbase = wbase - jnp.where(is_j, N_PAIRS, 0)

    def load_idx(ch, b):
        pltpu.sync_copy(idx_hbm.at[pl.ds(wbase + ch * CHUNK, CHUNK)],
                        idx_v.at[b])

    def fire_g(ch, b):
        for t in range(OPS):
            pltpu.async_copy(table_hbm.at[idx_v.at[b, pl.ds(t * OPB, OPB)]],
                             rows[b].at[pl.ds(t * OPB, OPB)], gsem[b])

    def drain_g(b):
        for t in range(OPS):
            pltpu.make_async_copy(
                table_hbm.at[idx_v.at[b, pl.ds(t * OPB, OPB)]],
                rows[b].at[pl.ds(t * OPB, OPB)], gsem[b]).wait()

    def wb_dst(ch):
        return out_hbm.at[pl.ds(pbase + ch * CHUNK, CHUNK), pl.ds(col, D)]

    def fire_wb(ch, b):
        pltpu.async_copy(rows[b], wb_dst(ch), wsem[b])

    def wait_wb(ch, b):
        pltpu.make_async_copy(rows[b], wb_dst(ch), wsem[b]).wait()

    load_idx(0, 0)
    fire_g(0, 0)

    @pl.loop(0, (G_CHUNKS - 1) // 2)
    def _pair(g):
        ch0 = 2 * g

        @pl.when(g > 0)
        def _():
            wait_wb(ch0 - 1, 1)

        load_idx(ch0 + 1, 1)
        fire_g(ch0 + 1, 1)
        drain_g(0)
        fire_wb(ch0, 0)

        wait_wb(ch0, 0)
        load_idx(ch0 + 2, 0)
        fire_g(ch0 + 2, 0)
        drain_g(1)
        fire_wb(ch0 + 1, 1)

    wait_wb(G_CHUNKS - 2, 1)
    drain_g(0)
    fire_wb(G_CHUNKS - 1, 0)
    wait_wb(G_CHUNKS - 1, 0)


def _gather(table, idx):
    k = pl.kernel(
        _gather_body,
        out_type=jax.ShapeDtypeStruct((N_PAIRS, 2 * D), jnp.float32),
        mesh=_mesh(),
        compiler_params=pltpu.CompilerParams(use_tc_tiling_on_sc=False),
        scratch_types=[
            pltpu.VMEM((2, CHUNK), jnp.int32),
            pltpu.VMEM((CHUNK, D), jnp.float32),
            pltpu.VMEM((CHUNK, D), jnp.float32),
            pltpu.SemaphoreType.DMA,
            pltpu.SemaphoreType.DMA,
            pltpu.SemaphoreType.DMA,
            pltpu.SemaphoreType.DMA,
        ],
    )
    return k(table, idx)


def _ffn_body(g_ref, bt_ref, wpi_ref, bpi_ref, e_ref, wii10_ref, o_ref):
    x = g_ref[...].astype(jnp.bfloat16)
    y = jnp.tanh(jnp.dot(x, wpi_ref[...], preferred_element_type=jnp.float32)
                 + bpi_ref[...])
    bexp = lax.dot_general(bt_ref[...].astype(jnp.bfloat16), e_ref[...],
                           (((0,), (0,)), ((), ())),
                           preferred_element_type=jnp.float32)
    yp = (y * bexp).astype(jnp.bfloat16)
    o = jnp.tanh(jnp.dot(yp, wii10_ref[...], preferred_element_type=jnp.float32))
    o_ref[:, 0:D] = o


def _ffn(g, basis_t, wpi, bpi, e, wii10):
    blk = 1280
    grid = N_PAIRS // blk
    return pl.pallas_call(
        _ffn_body,
        grid=(grid,),
        in_specs=[
            pl.BlockSpec((blk, 2 * D), lambda i: (i, 0)),
            pl.BlockSpec((NB, blk), lambda i: (0, i)),
            pl.BlockSpec((2 * D, NB * D), lambda i: (0, 0)),
            pl.BlockSpec((1, NB * D), lambda i: (0, 0)),
            pl.BlockSpec((NB, NB * D), lambda i: (0, 0)),
            pl.BlockSpec((NB * D, D), lambda i: (0, 0)),
        ],
        out_specs=pl.BlockSpec((blk, 2 * D), lambda i: (i, 0)),
        out_shape=jax.ShapeDtypeStruct((N_PAIRS, 2 * D), jnp.float32),
    )(g, basis_t, wpi, bpi, e, wii10)


def _scatter_body(ii_hbm, inter_hbm, zeros_hbm, out_hbm,
                  idx_raw, rows_a, rows_b, opb_a, opb_b,
                  acc, lsem_a, lsem_b, asem_a, asem_b):
    rows = (rows_a, rows_b)
    opb = (opb_a, opb_b)
    lsem = (lsem_a, lsem_b)
    asem = (asem_a, asem_b)
    c = lax.axis_index("c")
    s = lax.axis_index("s")
    base_atom = c * HALF

    pltpu.sync_copy(zeros_hbm.at[pl.ds(s * T_ROWS, T_ROWS)],
                    acc.at[pl.ds(s * T_ROWS, T_ROWS)])

    @pl.when(s == 0)
    def _():
        pltpu.sync_copy(zeros_hbm.at[pl.ds(HALF, 8)], acc.at[pl.ds(HALF, 8)])

    plsc.subcore_barrier()

    def srcs(ch, b):
        pbase = s * S_PER_T + ch * SCH
        return (ii_hbm.at[pl.ds(pbase, SCH)],
                inter_hbm.at[pl.ds(pbase, SCH), pl.ds(0, D)])

    def fire_loads(ch, b):
        isrc, rsrc = srcs(ch, b)
        pltpu.async_copy(isrc, idx_raw.at[b], lsem[b])
        pltpu.async_copy(rsrc, rows[b], lsem[b])

    def wait_loads(ch, b):
        isrc, rsrc = srcs(ch, b)
        pltpu.make_async_copy(isrc, idx_raw.at[b], lsem[b]).wait()
        pltpu.make_async_copy(rsrc, rows[b], lsem[b]).wait()

    def transform(b):
        for i in range(SCH // 16):
            v = idx_raw[b, pl.ds(i * 16, 16)]
            local = v - base_atom
            ok = (local >= 0) & (local < HALF)
            idx2 = jnp.where(ok, local, HALF)
            opb[b][pl.ds(i * 16, 16)] = idx2

    def fire_adds(b):
        pltpu.async_copy(rows[b], acc.at[opb[b]], asem[b], add=True)

    def drain_adds(b):
        pltpu.make_async_copy(rows[b], acc.at[opb[b]], asem[b]).wait()

    fire_loads(0, 0)

    @pl.loop(0, (S_CHUNKS - 1) // 2)
    def _pair(g):
        ch0 = 2 * g
        wait_loads(ch0, 0)

        @pl.when(g > 0)
        def _():
            drain_adds(1)

        fire_loads(ch0 + 1, 1)
        transform(0)
        fire_adds(0)

        wait_loads(ch0 + 1, 1)
        drain_adds(0)
        fire_loads(ch0 + 2, 0)
        transform(1)
        fire_adds(1)

    wait_loads(S_CHUNKS - 1, 0)
    drain_adds(1)
    transform(0)
    fire_adds(0)
    drain_adds(0)

    plsc.subcore_barrier()

    is_last = jnp.logical_and(c == NC - 1, s == NS - 1)

    @pl.when(jnp.logical_not(is_last))
    def _():
        pltpu.sync_copy(acc.at[pl.ds(s * T_ROWS, T_ROWS)],
                        out_hbm.at[pl.ds(base_atom + s * T_ROWS, T_ROWS)])

    @pl.when(is_last)
    def _():
        pltpu.sync_copy(acc.at[pl.ds((NS - 1) * T_ROWS, LAST_ROWS)],
                        out_hbm.at[pl.ds(HALF + (NS - 1) * T_ROWS, LAST_ROWS)])


def _scatter(ind_i, inter, zeros):
    k = pl.kernel(
        _scatter_body,
        out_type=jax.ShapeDtypeStruct((N_ATOMS, D), jnp.float32),
        mesh=_mesh(),
        compiler_params=pltpu.CompilerParams(use_tc_tiling_on_sc=False),
        scratch_types=[
            pltpu.VMEM((2, SCH), jnp.int32),
            pltpu.VMEM((SCH, D), jnp.float32),
            pltpu.VMEM((SCH, D), jnp.float32),
            pltpu.VMEM((SCH,), jnp.int32),
            pltpu.VMEM((SCH,), jnp.int32),
            pltpu.VMEM_SHARED((ACC_ROWS, D), jnp.float32),
            pltpu.SemaphoreType.DMA,
            pltpu.SemaphoreType.DMA,
            pltpu.SemaphoreType.DMA,
            pltpu.SemaphoreType.DMA,
        ],
    )
    return k(ind_i, inter, zeros)


def kernel(ind_2, prop, basis, W_pp1, b_pp1, W_pp2, b_pp2, W_pi, b_pi, W_ii):
    wpi = (W_pi.reshape(2 * D, D, NB).transpose(0, 2, 1)
           .reshape(2 * D, NB * D).astype(jnp.bfloat16))
    bpi = b_pi.reshape(D, NB).T.reshape(1, NB * D)
    w1 = W_pp1.astype(jnp.bfloat16)
    w2 = W_pp2.astype(jnp.bfloat16)
    b1 = b_pp1.reshape(1, D)
    b2 = b_pp2.reshape(1, D)
    e = jnp.repeat(jnp.eye(NB, dtype=jnp.bfloat16), D, axis=1)
    wii10 = jnp.tile(W_ii, (NB, 1)).astype(jnp.bfloat16)

    ind_flat = ind_2.T.reshape(ROWS_G)
    zeros = jnp.zeros((ACC_ROWS, D), jnp.float32)

    p = _pp(prop, w1, b1, w2, b2)
    g = _gather(p, ind_flat)
    inter = _ffn(g, basis.T, wpi, bpi, e, wii10)
    out = _scatter(ind_2[:, 0], inter, zeros)
    return out


# speedup vs baseline: 6.4526x; 1.1961x over previous
import jax
import jax.numpy as jnp
from jax import lax
from jax.experimental import pallas as pl
from jax.experimental.pallas import tpu as pltpu
from jax.experimental.pallas import tpu_sc as plsc

N_ATOMS = 50000
N_PAIRS = 800000
D = 64
NB = 10

NC = 2
NS = 16
NW = NC * NS

H0 = 403200
H1 = N_PAIRS - H0

CHUNK = 400
OPS = 5
OPB = 80

SCH = 80

HALF = 25088
T_ROWS = HALF // NS
ACC_ROWS = HALF + 8
LAST_ROWS = N_ATOMS - HALF - 15 * T_ROWS
FFN_BLK = 1280


def _mesh():
    return plsc.VectorSubcoreMesh(
        core_axis_name="c", subcore_axis_name="s", num_cores=NC, num_subcores=NS
    )


def _pp_body(x_ref, w1_ref, b1_ref, w2_ref, b2_ref, o_ref):
    x = x_ref[...].astype(jnp.bfloat16)
    h = jnp.tanh(jnp.dot(x, w1_ref[...], preferred_element_type=jnp.float32)
                 + b1_ref[...])
    p = jnp.tanh(jnp.dot(h.astype(jnp.bfloat16), w2_ref[...],
                         preferred_element_type=jnp.float32) + b2_ref[...])
    o_ref[...] = p


def _pp(prop, w1, b1, w2, b2):
    blk = 2000
    grid = N_ATOMS // blk
    return pl.pallas_call(
        _pp_body,
        grid=(grid,),
        in_specs=[
            pl.BlockSpec((blk, D), lambda i: (i, 0)),
            pl.BlockSpec((D, D), lambda i: (0, 0)),
            pl.BlockSpec((1, D), lambda i: (0, 0)),
            pl.BlockSpec((D, D), lambda i: (0, 0)),
            pl.BlockSpec((1, D), lambda i: (0, 0)),
        ],
        out_specs=pl.BlockSpec((blk, D), lambda i: (i, 0)),
        out_shape=jax.ShapeDtypeStruct((N_ATOMS, D), jnp.float32),
    )(prop, w1, b1, w2, b2)


def _gather_half(table, ii, jj, h, off):
    rpw = h // 16
    n_ch = rpw // CHUNK

    def body(table_hbm, ii_hbm, jj_hbm, out_hbm,
             idx_v, rows_a, rows_b, gsem_a, gsem_b, wsem_a, wsem_b):
        rows = (rows_a, rows_b)
        gsem = (gsem_a, gsem_b)
        wsem = (wsem_a, wsem_b)
        wid = lax.axis_index("s") * NC + lax.axis_index("c")
        wbase = wid * rpw
        is_j = wbase >= h
        col = jnp.where(is_j, D, 0)
        pbase = wbase - jnp.where(is_j, h, 0)

        def load_idx(ch, b):
            src = off + pbase + ch * CHUNK

            @pl.when(is_j)
            def _():
                pltpu.sync_copy(jj_hbm.at[pl.ds(src, CHUNK)], idx_v.at[b])

            @pl.when(jnp.logical_not(is_j))
            def _():
                pltpu.sync_copy(ii_hbm.at[pl.ds(src, CHUNK)], idx_v.at[b])

        def fire_g(b):
            for t in range(OPS):
                pltpu.async_copy(
                    table_hbm.at[idx_v.at[b, pl.ds(t * OPB, OPB)]],
                    rows[b].at[pl.ds(t * OPB, OPB)], gsem[b])

        def drain_g(b):
            for t in range(OPS):
                pltpu.make_async_copy(
                    table_hbm.at[idx_v.at[b, pl.ds(t * OPB, OPB)]],
                    rows[b].at[pl.ds(t * OPB, OPB)], gsem[b]).wait()

        def wb_dst(ch):
            return out_hbm.at[pl.ds(pbase + ch * CHUNK, CHUNK), pl.ds(col, D)]

        def fire_wb(ch, b):
            pltpu.async_copy(rows[b], wb_dst(ch), wsem[b])

        def wait_wb(ch, b):
            pltpu.make_async_copy(rows[b], wb_dst(ch), wsem[b]).wait()

        load_idx(0, 0)
        fire_g(0)

        n_loop = (n_ch - 1) // 2 if n_ch % 2 else (n_ch - 2) // 2

        @pl.loop(0, n_loop)
        def _pair(g):
            ch0 = 2 * g

            @pl.when(g > 0)
            def _():
                wait_wb(ch0 - 1, 1)

            load_idx(ch0 + 1, 1)
            fire_g(1)
            drain_g(0)
            fire_wb(ch0, 0)

            wait_wb(ch0, 0)
            load_idx(ch0 + 2, 0)
            fire_g(0)
            drain_g(1)
            fire_wb(ch0 + 1, 1)

        if n_ch % 2:
            wait_wb(n_ch - 2, 1)
            drain_g(0)
            fire_wb(n_ch - 1, 0)
            wait_wb(n_ch - 1, 0)
        else:
            wait_wb(n_ch - 3, 1)
            load_idx(n_ch - 1, 1)
            fire_g(1)
            drain_g(0)
            fire_wb(n_ch - 2, 0)
            wait_wb(n_ch - 2, 0)
            drain_g(1)
            fire_wb(n_ch - 1, 1)
            wait_wb(n_ch - 1, 1)

    k = pl.kernel(
        body,
        out_type=jax.ShapeDtypeStruct((h, 2 * D), jnp.float32),
        mesh=_mesh(),
        compiler_params=pltpu.CompilerParams(use_tc_tiling_on_sc=False),
        scratch_types=[
            pltpu.VMEM((2, CHUNK), jnp.int32),
            pltpu.VMEM((CHUNK, D), jnp.float32),
            pltpu.VMEM((CHUNK, D), jnp.float32),
            pltpu.SemaphoreType.DMA,
            pltpu.SemaphoreType.DMA,
            pltpu.SemaphoreType.DMA,
            pltpu.SemaphoreType.DMA,
        ],
    )
    return k(table, ii, jj)


def _ffn_body(g_ref, bt_ref, wpi_ref, bpi_ref, e_ref, wii10_ref, o_ref):
    x = g_ref[...].astype(jnp.bfloat16)
    y = jnp.tanh(jnp.dot(x, wpi_ref[...], preferred_element_type=jnp.float32)
                 + bpi_ref[...])
    bexp = lax.dot_general(bt_ref[...].astype(jnp.bfloat16), e_ref[...],
                           (((0,), (0,)), ((), ())),
                           preferred_element_type=jnp.float32)
    yp = (y * bexp).astype(jnp.bfloat16)
    o = jnp.tanh(jnp.dot(yp, wii10_ref[...], preferred_element_type=jnp.float32))
    o_ref[:, 0:D] = o


def _ffn_half(g, basis_t, wpi, bpi, e, wii10, h, boff):
    grid = h // FFN_BLK
    return pl.pallas_call(
        _ffn_body,
        grid=(grid,),
        in_specs=[
            pl.BlockSpec((FFN_BLK, 2 * D), lambda i: (i, 0)),
            pl.BlockSpec((NB, FFN_BLK), lambda i: (0, i + boff)),
            pl.BlockSpec((2 * D, NB * D), lambda i: (0, 0)),
            pl.BlockSpec((1, NB * D), lambda i: (0, 0)),
            pl.BlockSpec((NB, NB * D), lambda i: (0, 0)),
            pl.BlockSpec((NB * D, D), lambda i: (0, 0)),
        ],
        out_specs=pl.BlockSpec((FFN_BLK, 2 * D), lambda i: (i, 0)),
        out_shape=jax.ShapeDtypeStruct((h, 2 * D), jnp.float32),
    )(g, basis_t, wpi, bpi, e, wii10)


def _scatter_half(ii, inter, init, h, off):
    ppt = h // NS
    n_ch = ppt // SCH

    def body(ii_hbm, inter_hbm, init_hbm, out_hbm,
             idx_raw, rows_a, rows_b, opb_a, opb_b,
             acc, lsem_a, lsem_b, asem_a, asem_b):
        rows = (rows_a, rows_b)
        opb = (opb_a, opb_b)
        lsem = (lsem_a, lsem_b)
        asem = (asem_a, asem_b)
        c = lax.axis_index("c")
        s = lax.axis_index("s")
        base_atom = c * HALF
        is_last = jnp.logical_and(c == NC - 1, s == NS - 1)

        @pl.when(jnp.logical_not(is_last))
        def _():
            pltpu.sync_copy(init_hbm.at[pl.ds(base_atom + s * T_ROWS, T_ROWS)],
                            acc.at[pl.ds(s * T_ROWS, T_ROWS)])

        @pl.when(is_last)
        def _():
            pltpu.sync_copy(
                init_hbm.at[pl.ds(HALF + (NS - 1) * T_ROWS, LAST_ROWS)],
                acc.at[pl.ds((NS - 1) * T_ROWS, LAST_ROWS)])

        plsc.subcore_barrier()

        def srcs(ch, b):
            pbase = s * ppt + ch * SCH
            return (ii_hbm.at[pl.ds(off + pbase, SCH)],
                    inter_hbm.at[pl.ds(pbase, SCH), pl.ds(0, D)])

        def fire_loads(ch, b):
            isrc, rsrc = srcs(ch, b)
            pltpu.async_copy(isrc, idx_raw.at[b], lsem[b])
            pltpu.async_copy(rsrc, rows[b], lsem[b])

        def wait_loads(ch, b):
            isrc, rsrc = srcs(ch, b)
            pltpu.make_async_copy(isrc, idx_raw.at[b], lsem[b]).wait()
            pltpu.make_async_copy(rsrc, rows[b], lsem[b]).wait()

        def transform(b):
            for i in range(SCH // 16):
                v = idx_raw[b, pl.ds(i * 16, 16)]
                local = v - base_atom
                ok = (local >= 0) & (local < HALF)
                idx2 = jnp.where(ok, local, HALF)
                opb[b][pl.ds(i * 16, 16)] = idx2

        def fire_adds(b):
            pltpu.async_copy(rows[b], acc.at[opb[b]], asem[b], add=True)

        def drain_adds(b):
            pltpu.make_async_copy(rows[b], acc.at[opb[b]], asem[b]).wait()

        fire_loads(0, 0)

        n_loop = (n_ch - 1) // 2 if n_ch % 2 else (n_ch - 2) // 2

        @pl.loop(0, n_loop)
        def _pair(g):
            ch0 = 2 * g
            wait_loads(ch0, 0)

            @pl.when(g > 0)
            def _():
                drain_adds(1)

            fire_loads(ch0 + 1, 1)
            transform(0)
            fire_adds(0)

            wait_loads(ch0 + 1, 1)
            drain_adds(0)
            fire_loads(ch0 + 2, 0)
            transform(1)
            fire_adds(1)

        if n_ch % 2:
            wait_loads(n_ch - 1, 0)
            drain_adds(1)
            transform(0)
            fire_adds(0)
            drain_adds(0)
        else:
            wait_loads(n_ch - 2, 0)
            drain_adds(1)
            fire_loads(n_ch - 1, 1)
            transform(0)
            fire_adds(0)
            wait_loads(n_ch - 1, 1)
            drain_adds(0)
            transform(1)
            fire_adds(1)
            drain_adds(1)

        plsc.subcore_barrier()

        @pl.when(jnp.logical_not(is_last))
        def _():
            pltpu.sync_copy(acc.at[pl.ds(s * T_ROWS, T_ROWS)],
                            out_hbm.at[pl.ds(base_atom + s * T_ROWS, T_ROWS)])

        @pl.when(is_last)
        def _():
            pltpu.sync_copy(
                acc.at[pl.ds((NS - 1) * T_ROWS, LAST_ROWS)],
                out_hbm.at[pl.ds(HALF + (NS - 1) * T_ROWS, LAST_ROWS)])

    k = pl.kernel(
        body,
        out_type=jax.ShapeDtypeStruct((N_ATOMS, D), jnp.float32),
        mesh=_mesh(),
        compiler_params=pltpu.CompilerParams(use_tc_tiling_on_sc=False),
        scratch_types=[
            pltpu.VMEM((2, SCH), jnp.int32),
            pltpu.VMEM((SCH, D), jnp.float32),
            pltpu.VMEM((SCH, D), jnp.float32),
            pltpu.VMEM((SCH,), jnp.int32),
            pltpu.VMEM((SCH,), jnp.int32),
            pltpu.VMEM_SHARED((ACC_ROWS, D), jnp.float32),
            pltpu.SemaphoreType.DMA,
            pltpu.SemaphoreType.DMA,
            pltpu.SemaphoreType.DMA,
            pltpu.SemaphoreType.DMA,
        ],
    )
    return k(ii, inter, init)


def kernel(ind_2, prop, basis, W_pp1, b_pp1, W_pp2, b_pp2, W_pi, b_pi, W_ii):
    wpi = (W_pi.reshape(2 * D, D, NB).transpose(0, 2, 1)
           .reshape(2 * D, NB * D).astype(jnp.bfloat16))
    bpi = b_pi.reshape(D, NB).T.reshape(1, NB * D)
    w1 = W_pp1.astype(jnp.bfloat16)
    w2 = W_pp2.astype(jnp.bfloat16)
    b1 = b_pp1.reshape(1, D)
    b2 = b_pp2.reshape(1, D)
    e = jnp.repeat(jnp.eye(NB, dtype=jnp.bfloat16), D, axis=1)
    wii10 = jnp.tile(W_ii, (NB, 1)).astype(jnp.bfloat16)

    ind_t = ind_2.T.reshape(2 * N_PAIRS)
    ii = ind_t[:N_PAIRS]
    jj = ind_t[N_PAIRS:]
    bt = basis.T
    zeros = jnp.zeros((N_ATOMS, D), jnp.float32)

    p = _pp(prop, w1, b1, w2, b2)
    g0 = _gather_half(p, ii, jj, H0, 0)
    g1 = _gather_half(p, ii, jj, H1, H0)
    inter0 = _ffn_half(g0, bt, wpi, bpi, e, wii10, H0, 0)
    inter1 = _ffn_half(g1, bt, wpi, bpi, e, wii10, H1, H0 // FFN_BLK)
    out0 = _scatter_half(ii, inter0, zeros, H0, 0)
    out = _scatter_half(ii, inter1, out0, H1, H0)
    return out
